# Initial kernel scaffold; baseline (speedup 1.0000x reference)
import jax
import jax.numpy as jnp
from jax.experimental import pallas as pl


def kernel(H, E, r_embed, ht, queries, layers):
    raise NotImplementedError("write your pallas kernel here")


# speedup vs baseline: 3.3672x; 3.3672x over previous
import functools

import jax
import jax.numpy as jnp
from jax import lax
from jax.experimental import pallas as pl
from jax.experimental.pallas import tpu as pltpu
from jax.experimental.pallas import tpu_sc as plsc

D = 128
NS = 16
NC = 2
GR = 4
EG = GR * 128
GR_R = 2
EG_R = GR_R * 128


def _sc_mesh():
    return plsc.VectorSubcoreMesh(core_axis_name="c", subcore_axis_name="s")


def _counts_call(t2d, h2d, zc):
    r_rows, _ = t2d.shape
    n = zc.shape[0] * NS
    ng = r_rows
    ngt = (ng + NS - 1) // NS
    nper = n // NS

    def body(t2d_ref, h2d_ref, zc_ref, ct_ref, ch_ref, acc, idxb, ones, sem):
        cid = lax.axis_index("c")
        sid = lax.axis_index("s")

        def fill(i, _):
            for jj in range(8):
                ones[i, pl.ds(jj * 16, 16)] = jnp.full((16,), 1.0, jnp.float32)
            return 0

        lax.fori_loop(0, 128, fill, 0)
        pltpu.sync_copy(zc_ref, acc.at[pl.ds(sid * nper, nper)])
        plsc.subcore_barrier()

        def stream(idx2d_ref):
            def grp(i, _):
                g = i * NS + sid

                @pl.when(g < ng)
                def _():
                    pltpu.sync_copy(idx2d_ref.at[pl.ds(g, 1)], idxb)
                    pltpu.sync_copy(ones, acc.at[idxb.at[0]], add=True)

                return 0

            lax.fori_loop(0, ngt, grp, 0)

        @pl.when(cid == 0)
        def _():
            stream(t2d_ref)

        @pl.when(cid == 1)
        def _():
            stream(h2d_ref)

        plsc.subcore_barrier()

        @pl.when(cid == 0)
        def _():
            pltpu.sync_copy(
                acc.at[pl.ds(sid * nper, nper)], ct_ref.at[pl.ds(sid * nper, nper)]
            )

        @pl.when(cid == 1)
        def _():
            pltpu.sync_copy(
                acc.at[pl.ds(sid * nper, nper)], ch_ref.at[pl.ds(sid * nper, nper)]
            )

    out = jax.ShapeDtypeStruct((n, 128), jnp.float32)
    return pl.kernel(
        body,
        out_type=(out, out),
        mesh=_sc_mesh(),
        compiler_params=pltpu.CompilerParams(use_tc_tiling_on_sc=False),
        scratch_types=[
            pltpu.VMEM_SHARED((n, 128), jnp.float32),
            pltpu.VMEM((1, 128), jnp.int32),
            pltpu.VMEM((128, 128), jnp.float32),
            pltpu.SemaphoreType.DMA,
        ],
    )(t2d, h2d, zc)


def _gather_c_call(g1, g3, h2d, t2d):
    n, d = g1.shape
    r_rows, _ = h2d.shape
    m = r_rows * 128
    ng = r_rows // GR
    ngt = (ng + NS - 1) // NS

    def body(g1_ref, g3_ref, h2d_ref, t2d_ref, c1_ref, c2_ref, idxb, rows, sem):
        cid = lax.axis_index("c")
        sid = lax.axis_index("s")

        def stream(table_ref, idx2d_ref, out_ref):
            def grp(i, _):
                g = i * NS + sid

                @pl.when(g < ng)
                def _():
                    rb = g * GR
                    pltpu.sync_copy(idx2d_ref.at[pl.ds(rb, GR)], idxb)
                    cps = [
                        pltpu.async_copy(
                            table_ref.at[idxb.at[j]],
                            rows.at[pl.ds(j * 128, 128)],
                            sem,
                        )
                        for j in range(GR)
                    ]
                    for cp in cps:
                        cp.wait()
                    pltpu.sync_copy(rows, out_ref.at[pl.ds(rb * 128, EG)])

                return 0

            lax.fori_loop(0, ngt, grp, 0)

        @pl.when(cid == 0)
        def _():
            stream(g1_ref, h2d_ref, c1_ref)

        @pl.when(cid == 1)
        def _():
            stream(g3_ref, t2d_ref, c2_ref)

    out = jax.ShapeDtypeStruct((m, d), jnp.float32)
    return pl.kernel(
        body,
        out_type=(out, out),
        mesh=_sc_mesh(),
        compiler_params=pltpu.CompilerParams(use_tc_tiling_on_sc=False),
        scratch_types=[
            pltpu.VMEM((GR, 128), jnp.int32),
            pltpu.VMEM((EG, d), jnp.float32),
            pltpu.SemaphoreType.DMA,
        ],
    )(g1, g3, h2d, t2d)


def _reduce_s_call(en, t2d, h2d, zrows):
    m, d = en.shape
    r_rows, _ = t2d.shape
    n = zrows.shape[0] * NS
    ng = r_rows // GR_R
    ngt = (ng + NS - 1) // NS
    nper = n // NS

    def body(en_ref, t2d_ref, h2d_ref, z_ref, sf_ref, sb_ref, acc, idxb, rows, sem):
        cid = lax.axis_index("c")
        sid = lax.axis_index("s")
        pltpu.sync_copy(z_ref, acc.at[pl.ds(sid * nper, nper)])
        plsc.subcore_barrier()

        def stream(idx2d_ref):
            def grp(i, _):
                g = i * NS + sid

                @pl.when(g < ng)
                def _():
                    rb = g * GR_R
                    pltpu.sync_copy(idx2d_ref.at[pl.ds(rb, GR_R)], idxb)
                    pltpu.sync_copy(en_ref.at[pl.ds(rb * 128, EG_R)], rows)
                    for j in range(GR_R):
                        pltpu.sync_copy(
                            rows.at[pl.ds(j * 128, 128)],
                            acc.at[idxb.at[j]],
                            add=True,
                        )

                return 0

            lax.fori_loop(0, ngt, grp, 0)

        @pl.when(cid == 0)
        def _():
            stream(t2d_ref)

        @pl.when(cid == 1)
        def _():
            stream(h2d_ref)

        plsc.subcore_barrier()

        @pl.when(cid == 0)
        def _():
            pltpu.sync_copy(
                acc.at[pl.ds(sid * nper, nper)], sf_ref.at[pl.ds(sid * nper, nper)]
            )

        @pl.when(cid == 1)
        def _():
            pltpu.sync_copy(
                acc.at[pl.ds(sid * nper, nper)], sb_ref.at[pl.ds(sid * nper, nper)]
            )

    out = jax.ShapeDtypeStruct((n, d), jnp.float32)
    return pl.kernel(
        body,
        out_type=(out, out),
        mesh=_sc_mesh(),
        compiler_params=pltpu.CompilerParams(use_tc_tiling_on_sc=False),
        scratch_types=[
            pltpu.VMEM_SHARED((n, d), jnp.float32),
            pltpu.VMEM((GR_R, 128), jnp.int32),
            pltpu.VMEM((EG_R, d), jnp.float32),
            pltpu.SemaphoreType.DMA,
        ],
    )(en, t2d, h2d, zrows)


def _reduce_t_call(h, h2d, t2d, zrows):
    n, d = h.shape
    r_rows, _ = h2d.shape
    ng = r_rows // GR_R
    ngt = (ng + NS - 1) // NS
    nper = n // NS

    def body(h_ref, h2d_ref, t2d_ref, z_ref, tf_ref, tb_ref, acc, gidxb, sidxb, rows, sem):
        cid = lax.axis_index("c")
        sid = lax.axis_index("s")
        pltpu.sync_copy(z_ref, acc.at[pl.ds(sid * nper, nper)])
        plsc.subcore_barrier()

        def stream(gather_idx2d_ref, scatter_idx2d_ref):
            def grp(i, _):
                g = i * NS + sid

                @pl.when(g < ng)
                def _():
                    rb = g * GR_R
                    pltpu.sync_copy(gather_idx2d_ref.at[pl.ds(rb, GR_R)], gidxb)
                    pltpu.sync_copy(scatter_idx2d_ref.at[pl.ds(rb, GR_R)], sidxb)
                    cps = [
                        pltpu.async_copy(
                            h_ref.at[gidxb.at[j]],
                            rows.at[pl.ds(j * 128, 128)],
                            sem,
                        )
                        for j in range(GR_R)
                    ]
                    for cp in cps:
                        cp.wait()
                    for j in range(GR_R):
                        pltpu.sync_copy(
                            rows.at[pl.ds(j * 128, 128)],
                            acc.at[sidxb.at[j]],
                            add=True,
                        )

                return 0

            lax.fori_loop(0, ngt, grp, 0)

        @pl.when(cid == 0)
        def _():
            stream(h2d_ref, t2d_ref)

        @pl.when(cid == 1)
        def _():
            stream(t2d_ref, h2d_ref)

        plsc.subcore_barrier()

        @pl.when(cid == 0)
        def _():
            pltpu.sync_copy(
                acc.at[pl.ds(sid * nper, nper)], tf_ref.at[pl.ds(sid * nper, nper)]
            )

        @pl.when(cid == 1)
        def _():
            pltpu.sync_copy(
                acc.at[pl.ds(sid * nper, nper)], tb_ref.at[pl.ds(sid * nper, nper)]
            )

    out = jax.ShapeDtypeStruct((n, d), jnp.float32)
    return pl.kernel(
        body,
        out_type=(out, out),
        mesh=_sc_mesh(),
        compiler_params=pltpu.CompilerParams(use_tc_tiling_on_sc=False),
        scratch_types=[
            pltpu.VMEM_SHARED((n, d), jnp.float32),
            pltpu.VMEM((GR_R, 128), jnp.int32),
            pltpu.VMEM((GR_R, 128), jnp.int32),
            pltpu.VMEM((EG_R, d), jnp.float32),
            pltpu.SemaphoreType.DMA,
        ],
    )(h, h2d, t2d, zrows)


def _leaky(x):
    return jnp.where(x >= 0, x, 0.01 * x)


def _ln(x, g, b):
    mu = jnp.mean(x, axis=-1, keepdims=True)
    var = jnp.mean((x - mu) ** 2, axis=-1, keepdims=True)
    return (x - mu) * lax.rsqrt(var + 1e-5) * g + b


def _proj_call(h, w13):
    n, d = h.shape
    bn = 1000

    def body(h_ref, w_ref, out_ref):
        out_ref[...] = jnp.dot(
            h_ref[...], w_ref[...], preferred_element_type=jnp.float32
        )

    return pl.pallas_call(
        body,
        grid=(n // bn,),
        in_specs=[
            pl.BlockSpec((bn, d), lambda i: (i, 0)),
            pl.BlockSpec((d, 2 * d), lambda i: (0, 0)),
        ],
        out_specs=pl.BlockSpec((bn, 2 * d), lambda i: (i, 0)),
        out_shape=jax.ShapeDtypeStruct((n, 2 * d), jnp.float32),
    )(h, w13)


def _edge_call(e, c1, c2, a2, be, ge, bee):
    m, d = e.shape
    bm = 3200

    def body(e_ref, c1_ref, c2_ref, a2_ref, be_ref, ge_ref, bee_ref, out_ref):
        ev = e_ref[...]
        pre = (
            c1_ref[...]
            + c2_ref[...]
            + jnp.dot(ev, a2_ref[...], preferred_element_type=jnp.float32)
            + be_ref[...]
        )
        u = _leaky(pre) + ev
        out_ref[...] = _ln(u, ge_ref[...], bee_ref[...])

    return pl.pallas_call(
        body,
        grid=(m // bm,),
        in_specs=[
            pl.BlockSpec((bm, d), lambda i: (i, 0)),
            pl.BlockSpec((bm, d), lambda i: (i, 0)),
            pl.BlockSpec((bm, d), lambda i: (i, 0)),
            pl.BlockSpec((d, d), lambda i: (0, 0)),
            pl.BlockSpec((1, d), lambda i: (0, 0)),
            pl.BlockSpec((1, d), lambda i: (0, 0)),
            pl.BlockSpec((1, d), lambda i: (0, 0)),
        ],
        out_specs=pl.BlockSpec((bm, d), lambda i: (i, 0)),
        out_shape=jax.ShapeDtypeStruct((m, d), jnp.float32),
    )(e, c1, c2, a2, be, ge, bee)


def _node_call(h, tf, sf, tb, sb, ct, ch, w4, bf, bb, gn, bn_, w13):
    n, d = h.shape
    bn = 1000

    def body(
        h_ref, tf_ref, sf_ref, tb_ref, sb_ref, ct_ref, ch_ref,
        w4_ref, bf_ref, bb_ref, gn_ref, bn_ref, w13_ref,
        h_out, g13_out,
    ):
        x = jnp.concatenate(
            [tf_ref[...], sf_ref[...], tb_ref[...], sb_ref[...]], axis=-1
        )
        agg = jnp.dot(x, w4_ref[...], preferred_element_type=jnp.float32)
        cnt_t = ct_ref[...][:, :1]
        cnt_h = ch_ref[...][:, :1]
        agg = agg + cnt_t * bf_ref[...] + cnt_h * bb_ref[...]
        agg = agg / (cnt_t + cnt_h)
        u = _leaky(agg) + h_ref[...]
        hn = _ln(u, gn_ref[...], bn_ref[...])
        h_out[...] = hn
        g13_out[...] = jnp.dot(hn, w13_ref[...], preferred_element_type=jnp.float32)

    return pl.pallas_call(
        body,
        grid=(n // bn,),
        in_specs=[
            pl.BlockSpec((bn, d), lambda i: (i, 0)),
            pl.BlockSpec((bn, d), lambda i: (i, 0)),
            pl.BlockSpec((bn, d), lambda i: (i, 0)),
            pl.BlockSpec((bn, d), lambda i: (i, 0)),
            pl.BlockSpec((bn, d), lambda i: (i, 0)),
            pl.BlockSpec((bn, d), lambda i: (i, 0)),
            pl.BlockSpec((bn, d), lambda i: (i, 0)),
            pl.BlockSpec((4 * d, d), lambda i: (0, 0)),
            pl.BlockSpec((1, d), lambda i: (0, 0)),
            pl.BlockSpec((1, d), lambda i: (0, 0)),
            pl.BlockSpec((1, d), lambda i: (0, 0)),
            pl.BlockSpec((1, d), lambda i: (0, 0)),
            pl.BlockSpec((d, 2 * d), lambda i: (0, 0)),
        ],
        out_specs=[
            pl.BlockSpec((bn, d), lambda i: (i, 0)),
            pl.BlockSpec((bn, 2 * d), lambda i: (i, 0)),
        ],
        out_shape=[
            jax.ShapeDtypeStruct((n, d), jnp.float32),
            jax.ShapeDtypeStruct((n, 2 * d), jnp.float32),
        ],
    )(h, tf, sf, tb, sb, ct, ch, w4, bf, bb, gn, bn_, w13)


def kernel(H, E, r_embed, ht, queries, layers):
    n, d = H.shape
    m = E.shape[0]
    r_rows = m // 128

    heads = ht[:, 0].astype(jnp.int32)
    tails = ht[:, 1].astype(jnp.int32)
    h2d = heads.reshape(r_rows, 128)
    t2d = tails.reshape(r_rows, 128)

    zrows = jnp.zeros((n // NS, d), jnp.float32)
    zc = jnp.zeros((n // NS, 128), jnp.float32)

    packed = []
    for lp in layers:
        wet = lp["We"].T
        packed.append(
            dict(
                a2=wet[d : 2 * d],
                w13=jnp.concatenate([wet[:d], wet[2 * d :]], axis=1),
                w4=jnp.concatenate(
                    [lp["Wf"].T[:d], lp["Wf"].T[d:], lp["Wb"].T[:d], lp["Wb"].T[d:]],
                    axis=0,
                ),
                be=lp["be"].reshape(1, d),
                ge=lp["ge"].reshape(1, d),
                bee=lp["bee"].reshape(1, d),
                bf=lp["bf"].reshape(1, d),
                bb=lp["bb"].reshape(1, d),
                gn=lp["gn"].reshape(1, d),
                bn=lp["bn"].reshape(1, d),
            )
        )

    ct, ch = _counts_call(t2d, h2d, zc)

    g13 = _proj_call(H, packed[0]["w13"])
    nlayers = len(layers)
    for li, lp in enumerate(packed):
        g1 = g13[:, :d]
        g3 = g13[:, d:]
        tf, tb = _reduce_t_call(H, h2d, t2d, zrows)
        c1, c2 = _gather_c_call(g1, g3, h2d, t2d)
        en = _edge_call(E, c1, c2, lp["a2"], lp["be"], lp["ge"], lp["bee"])
        sf, sb = _reduce_s_call(en, t2d, h2d, zrows)
        w13_next = packed[li + 1]["w13"] if li + 1 < nlayers else packed[li]["w13"]
        H, g13 = _node_call(
            H, tf, sf, tb, sb, ct, ch,
            lp["w4"], lp["bf"], lp["bb"], lp["gn"], lp["bn"], w13_next,
        )
        E = en
    return H
